# Initial kernel scaffold; baseline (speedup 1.0000x reference)
import jax
import jax.numpy as jnp
from jax.experimental import pallas as pl


def kernel(x, edge_index, edge_type, node_emb, w1, root1, b1, w2, root2, b2):
    raise NotImplementedError("write your pallas kernel here")


# speedup vs baseline: 8.1445x; 8.1445x over previous
import functools

import jax
import jax.numpy as jnp
from jax import lax
from jax.experimental import pallas as pl
from jax.experimental.pallas import tpu as pltpu
from jax.experimental.pallas import tpu_sc as plsc

_N = 10000
_E = 320000
_H = 128
_R = 8

_L = 16
_NC = 2
_NS = 16
_HC = _H // _L
_RN = _R * _N

_EP_ROWS = 2560
_E_PAD = _EP_ROWS * 128
_KB = 8
_ROWS_T = _EP_ROWS // _NS
_NSB = _ROWS_T // _KB
_ROWS_TC = _EP_ROWS // (_NC * _NS)
_NSB_C = _ROWS_TC // _KB
_ACC_ROWS = _RN + _L
_DUMP = _RN // _NS
_ZB = 500

_BN = 1000


def _sc_body(do_count, h_tab, src2d, seg2d, *rest):
    if do_count:
        (acc_out, cnt_out, acc_sp, zero_v, ones_v, src_v, seg_v, idx_v,
         rows_v, gsem, ssem) = rest
    else:
        (acc_out, acc_sp, zero_v, src_v, seg_v, idx_v, rows_v,
         gsem, ssem) = rest

    cid = lax.axis_index("c")
    sid = lax.axis_index("s")

    def init_zero(t, carry):
        zero_v[t] = jnp.zeros((_L,), jnp.float32)
        return carry

    lax.fori_loop(0, _ZB, init_zero, 0)

    if do_count:
        def init_ones(t, carry):
            ones_v[t] = jnp.ones((_L,), jnp.float32)
            return carry

        lax.fori_loop(0, 128, init_ones, 0)

    def zero_own_slice():
        for k in range(_DUMP // _ZB):
            pltpu.sync_copy(
                zero_v, acc_sp.at[pl.ds(sid * _DUMP + k * _ZB, _ZB)])

    zero_own_slice()
    plsc.subcore_barrier()

    if do_count:
        w = cid * _NS + sid

        def cnt_step(s, carry):
            rb = w * _ROWS_TC + s * _KB
            pltpu.sync_copy(seg2d.at[pl.ds(rb, _KB)], seg_v)
            descs = [
                pltpu.async_copy(ones_v, acc_sp.at[seg_v.at[j]], ssem,
                                 add=True)
                for j in range(_KB)
            ]
            for d in descs:
                d.wait()
            return carry

        lax.fori_loop(0, _NSB_C, cnt_step, 0)
        plsc.subcore_barrier()
        pltpu.sync_copy(
            acc_sp.at[pl.ds(sid * _DUMP, _DUMP)],
            cnt_out.at[pl.ds(cid * _RN + sid * _DUMP, _DUMP)])
        zero_own_slice()
        plsc.subcore_barrier()

    for k in range(_HC // _NC):
        c = cid * (_HC // _NC) + k
        c_off = c * _N

        def chunk_step(s, carry):
            rb = sid * _ROWS_T + s * _KB
            pltpu.sync_copy(src2d.at[pl.ds(rb, _KB)], src_v)
            pltpu.sync_copy(seg2d.at[pl.ds(rb, _KB)], seg_v)

            def addoff(t, c2):
                for jj in range(_KB):
                    idx_v[jj, pl.ds(t * _L, _L)] = (
                        src_v[jj, pl.ds(t * _L, _L)] + c_off)
                return c2

            lax.fori_loop(0, 128 // _L, addoff, 0)
            g = [
                pltpu.async_copy(h_tab.at[idx_v.at[j]],
                                 rows_v.at[pl.ds(j * 128, 128)], gsem)
                for j in range(_KB)
            ]
            for d in g:
                d.wait()
            sc = [
                pltpu.async_copy(rows_v.at[pl.ds(j * 128, 128)],
                                 acc_sp.at[seg_v.at[j]], ssem, add=True)
                for j in range(_KB)
            ]
            for d in sc:
                d.wait()
            return carry

        lax.fori_loop(0, _NSB, chunk_step, 0)
        plsc.subcore_barrier()
        pltpu.sync_copy(
            acc_sp.at[pl.ds(sid * _DUMP, _DUMP)],
            acc_out.at[pl.ds(c * _RN + sid * _DUMP, _DUMP)])
        if k < _HC // _NC - 1:
            zero_own_slice()
            plsc.subcore_barrier()


def _make_sc_kernel(do_count):
    mesh = plsc.VectorSubcoreMesh(
        core_axis_name="c", subcore_axis_name="s",
        num_cores=_NC, num_subcores=_NS)
    out_type = [jax.ShapeDtypeStruct((_HC * _RN, _L), jnp.float32)]
    scratch = [
        pltpu.VMEM_SHARED((_ACC_ROWS, _L), jnp.float32),
        pltpu.VMEM((_ZB, _L), jnp.float32),
    ]
    if do_count:
        out_type.append(jax.ShapeDtypeStruct((_NC * _RN, _L), jnp.float32))
        scratch.append(pltpu.VMEM((128, _L), jnp.float32))
    scratch += [
        pltpu.VMEM((_KB, 128), jnp.int32),
        pltpu.VMEM((_KB, 128), jnp.int32),
        pltpu.VMEM((_KB, 128), jnp.int32),
        pltpu.VMEM((_KB * 128, _L), jnp.float32),
        pltpu.SemaphoreType.DMA,
        pltpu.SemaphoreType.DMA,
    ]
    return pl.kernel(
        functools.partial(_sc_body, do_count),
        out_type=tuple(out_type),
        mesh=mesh,
        scratch_types=scratch,
        compiler_params=pltpu.CompilerParams(use_tc_tiling_on_sc=False),
    )


def _dense_body(h_ref, sums_ref, invt_ref, w_ref, root_ref, b_ref, o_ref):
    acc = jnp.dot(h_ref[...], root_ref[...],
                  preferred_element_type=jnp.float32) + b_ref[...]
    for r in range(_R):
        mean = sums_ref[r] * invt_ref[:, r:r + 1]
        acc = acc + jnp.dot(mean, w_ref[r],
                            preferred_element_type=jnp.float32)
    o_ref[...] = jnp.maximum(acc, 0.0)


def _dense_combine(h, sums, invt, w, root, b):
    grid = (_N // _BN,)
    return pl.pallas_call(
        _dense_body,
        grid=grid,
        in_specs=[
            pl.BlockSpec((_BN, _H), lambda i: (i, 0)),
            pl.BlockSpec((_R, _BN, _H), lambda i: (0, i, 0)),
            pl.BlockSpec((_BN, _R), lambda i: (i, 0)),
            pl.BlockSpec((_R, _H, _H), lambda i: (0, 0, 0)),
            pl.BlockSpec((_H, _H), lambda i: (0, 0)),
            pl.BlockSpec((1, _H), lambda i: (0, 0)),
        ],
        out_specs=pl.BlockSpec((_BN, _H), lambda i: (i, 0)),
        out_shape=jax.ShapeDtypeStruct((_N, _H), jnp.float32),
    )(h, sums, invt, w, root, b.reshape(1, _H))


_sc_agg_count = _make_sc_kernel(True)
_sc_agg = _make_sc_kernel(False)


def kernel(x, edge_index, edge_type, node_emb, w1, root1, b1, w2, root2, b2):
    src = edge_index[0]
    dst = edge_index[1]
    seg = edge_type * _N + dst
    pad = _E_PAD - _E
    src_p = jnp.concatenate([src, jnp.zeros((pad,), jnp.int32)])
    seg_p = jnp.concatenate([seg, jnp.full((pad,), _RN, jnp.int32)])
    src2d = src_p.reshape(_EP_ROWS, 128)
    seg2d = seg_p.reshape(_EP_ROWS, 128)

    h = jnp.take(node_emb, x, axis=0)

    invt = None
    for (w, root, b, first) in ((w1, root1, b1, True),
                                (w2, root2, b2, False)):
        h_tab = h.reshape(_N, _HC, _L).transpose(1, 0, 2).reshape(
            _HC * _N, _L)
        if first:
            acc, cnt2 = _sc_agg_count(h_tab, src2d, seg2d)
            cnt = cnt2[:_RN, 0] + cnt2[_RN:, 0]
            invt = (1.0 / jnp.maximum(cnt, 1.0)).reshape(_R, _N).T
        else:
            (acc,) = _sc_agg(h_tab, src2d, seg2d)
        sums = acc.reshape(_HC, _R, _N, _L).transpose(1, 2, 0, 3).reshape(
            _R, _N, _H)
        h = _dense_combine(h, sums, invt, w, root, b)
    return h


# speedup vs baseline: 12.3413x; 1.5153x over previous
import functools

import jax
import jax.numpy as jnp
from jax import lax
from jax.experimental import pallas as pl
from jax.experimental.pallas import tpu as pltpu
from jax.experimental.pallas import tpu_sc as plsc

_N = 10000
_E = 320000
_H = 128
_R = 8

_L = 16
_NC = 2
_NS = 16
_HC = _H // _L
_RN = _R * _N

_EP_ROWS = 2560
_E_PAD = _EP_ROWS * 128
_KB = 8
_ROWS_T = _EP_ROWS // _NS
_NSB = _ROWS_T // _KB
_ROWS_TC = _EP_ROWS // (_NC * _NS)
_NSB_C = _ROWS_TC // _KB
_ACC_ROWS = _RN + _L
_DUMP = _RN // _NS
_ZB = 500

_BN = 1000


def _sc_body(do_count, h_tab, src2d, seg2d, *rest):
    if do_count:
        (acc_out, cnt_out, acc_sp, zero_v, ones_v, src_a, src_b, seg_a,
         seg_b, rows_a, rows_b, lsem, gsem, ssem) = rest
    else:
        (acc_out, acc_sp, zero_v, src_a, src_b, seg_a, seg_b, rows_a,
         rows_b, lsem, gsem, ssem) = rest

    cid = lax.axis_index("c")
    sid = lax.axis_index("s")

    def init_zero(t, carry):
        zero_v[t] = jnp.zeros((_L,), jnp.float32)
        return carry

    lax.fori_loop(0, _ZB, init_zero, 0)

    if do_count:
        def init_ones(t, carry):
            ones_v[t] = jnp.ones((_L,), jnp.float32)
            return carry

        lax.fori_loop(0, 128, init_ones, 0)

    def zero_own_slice():
        for k in range(_DUMP // _ZB):
            pltpu.sync_copy(
                zero_v, acc_sp.at[pl.ds(sid * _DUMP + k * _ZB, _ZB)])

    zero_own_slice()
    plsc.subcore_barrier()

    if do_count:
        w = cid * _NS + sid

        def cnt_step(s, carry):
            rb = w * _ROWS_TC + s * _KB
            pltpu.sync_copy(seg2d.at[pl.ds(rb, _KB)], seg_a)
            descs = [
                pltpu.async_copy(ones_v, acc_sp.at[seg_a.at[j]], ssem,
                                 add=True)
                for j in range(_KB)
            ]
            for d in descs:
                d.wait()
            return carry

        lax.fori_loop(0, _NSB_C, cnt_step, 0)
        plsc.subcore_barrier()
        pltpu.sync_copy(
            acc_sp.at[pl.ds(sid * _DUMP, _DUMP)],
            cnt_out.at[pl.ds(cid * _RN + sid * _DUMP, _DUMP)])
        zero_own_slice()
        plsc.subcore_barrier()

    for k in range(_HC // _NC):
        c = cid * (_HC // _NC) + k
        tab_c = h_tab.at[pl.ds(c * _N, _N)]

        def pair_step(i, carry):
            ra = sid * _ROWS_T + (2 * i) * _KB
            rbb = ra + _KB
            la = [pltpu.async_copy(src2d.at[pl.ds(ra, _KB)], src_a, lsem),
                  pltpu.async_copy(seg2d.at[pl.ds(ra, _KB)], seg_a, lsem)]
            lb = [pltpu.async_copy(src2d.at[pl.ds(rbb, _KB)], src_b, lsem),
                  pltpu.async_copy(seg2d.at[pl.ds(rbb, _KB)], seg_b, lsem)]
            for d in la:
                d.wait()
            ga = [
                pltpu.async_copy(tab_c.at[src_a.at[j]],
                                 rows_a.at[pl.ds(j * 128, 128)], gsem)
                for j in range(_KB)
            ]
            for d in lb:
                d.wait()
            for d in ga:
                d.wait()
            sa = [
                pltpu.async_copy(rows_a.at[pl.ds(j * 128, 128)],
                                 acc_sp.at[seg_a.at[j]], ssem, add=True)
                for j in range(_KB)
            ]
            gb = [
                pltpu.async_copy(tab_c.at[src_b.at[j]],
                                 rows_b.at[pl.ds(j * 128, 128)], gsem)
                for j in range(_KB)
            ]
            for d in gb:
                d.wait()
            sb = [
                pltpu.async_copy(rows_b.at[pl.ds(j * 128, 128)],
                                 acc_sp.at[seg_b.at[j]], ssem, add=True)
                for j in range(_KB)
            ]
            for d in sa:
                d.wait()
            for d in sb:
                d.wait()
            return carry

        lax.fori_loop(0, _NSB // 2, pair_step, 0)
        plsc.subcore_barrier()
        pltpu.sync_copy(
            acc_sp.at[pl.ds(sid * _DUMP, _DUMP)],
            acc_out.at[pl.ds(sid * _DUMP, _DUMP), pl.ds(c * _L, _L)])
        if k < _HC // _NC - 1:
            zero_own_slice()
            plsc.subcore_barrier()


def _make_sc_kernel(do_count):
    mesh = plsc.VectorSubcoreMesh(
        core_axis_name="c", subcore_axis_name="s",
        num_cores=_NC, num_subcores=_NS)
    out_type = [jax.ShapeDtypeStruct((_RN, _H), jnp.float32)]
    scratch = [
        pltpu.VMEM_SHARED((_ACC_ROWS, _L), jnp.float32),
        pltpu.VMEM((_ZB, _L), jnp.float32),
    ]
    if do_count:
        out_type.append(jax.ShapeDtypeStruct((_NC * _RN, _L), jnp.float32))
        scratch.append(pltpu.VMEM((128, _L), jnp.float32))
    scratch += [
        pltpu.VMEM((_KB, 128), jnp.int32),
        pltpu.VMEM((_KB, 128), jnp.int32),
        pltpu.VMEM((_KB, 128), jnp.int32),
        pltpu.VMEM((_KB, 128), jnp.int32),
        pltpu.VMEM((_KB * 128, _L), jnp.float32),
        pltpu.VMEM((_KB * 128, _L), jnp.float32),
        pltpu.SemaphoreType.DMA,
        pltpu.SemaphoreType.DMA,
        pltpu.SemaphoreType.DMA,
    ]
    return pl.kernel(
        functools.partial(_sc_body, do_count),
        out_type=tuple(out_type),
        mesh=mesh,
        scratch_types=scratch,
        compiler_params=pltpu.CompilerParams(use_tc_tiling_on_sc=False),
    )


def _dense_body(h_ref, sums_ref, invt_ref, w_ref, root_ref, b_ref, o_ref,
                oc_ref):
    acc = jnp.dot(h_ref[...], root_ref[...],
                  preferred_element_type=jnp.float32) + b_ref[...]
    for r in range(_R):
        mean = sums_ref[r] * invt_ref[:, r:r + 1]
        acc = acc + jnp.dot(mean, w_ref[r],
                            preferred_element_type=jnp.float32)
    res = jnp.maximum(acc, 0.0)
    o_ref[...] = res
    for c in range(_HC):
        oc_ref[c] = res[:, c * _L:(c + 1) * _L]


def _dense_combine(h, sums, invt, w, root, b):
    grid = (_N // _BN,)
    return pl.pallas_call(
        _dense_body,
        grid=grid,
        in_specs=[
            pl.BlockSpec((_BN, _H), lambda i: (i, 0)),
            pl.BlockSpec((_R, _BN, _H), lambda i: (0, i, 0)),
            pl.BlockSpec((_BN, _R), lambda i: (i, 0)),
            pl.BlockSpec((_R, _H, _H), lambda i: (0, 0, 0)),
            pl.BlockSpec((_H, _H), lambda i: (0, 0)),
            pl.BlockSpec((1, _H), lambda i: (0, 0)),
        ],
        out_specs=[
            pl.BlockSpec((_BN, _H), lambda i: (i, 0)),
            pl.BlockSpec((_HC, _BN, _L), lambda i: (0, i, 0)),
        ],
        out_shape=[
            jax.ShapeDtypeStruct((_N, _H), jnp.float32),
            jax.ShapeDtypeStruct((_HC, _N, _L), jnp.float32),
        ],
    )(h, sums, invt, w, root, b.reshape(1, _H))


_sc_agg_count = _make_sc_kernel(True)
_sc_agg = _make_sc_kernel(False)


def kernel(x, edge_index, edge_type, node_emb, w1, root1, b1, w2, root2, b2):
    src = edge_index[0]
    dst = edge_index[1]
    seg = edge_type * _N + dst
    pad = _E_PAD - _E
    src_p = jnp.concatenate([src, jnp.zeros((pad,), jnp.int32)])
    seg_p = jnp.concatenate([seg, jnp.full((pad,), _RN, jnp.int32)])
    src2d = src_p.reshape(_EP_ROWS, 128)
    seg2d = seg_p.reshape(_EP_ROWS, 128)

    h = jnp.take(node_emb, x, axis=0)
    h_tab = h.reshape(_N, _HC, _L).transpose(1, 0, 2).reshape(_HC * _N, _L)

    invt = None
    for (w, root, b, first) in ((w1, root1, b1, True),
                                (w2, root2, b2, False)):
        if first:
            acc, cnt2 = _sc_agg_count(h_tab, src2d, seg2d)
            cnt = cnt2[:_RN, 0] + cnt2[_RN:, 0]
            invt = (1.0 / jnp.maximum(cnt, 1.0)).reshape(_R, _N).T
        else:
            (acc,) = _sc_agg(h_tab, src2d, seg2d)
        sums = acc.reshape(_R, _N, _H)
        h, h_chunks = _dense_combine(h, sums, invt, w, root, b)
        h_tab = h_chunks.reshape(_HC * _N, _L)
    return h


# speedup vs baseline: 12.9864x; 1.0523x over previous
import functools

import jax
import jax.numpy as jnp
from jax import lax
from jax.experimental import pallas as pl
from jax.experimental.pallas import tpu as pltpu
from jax.experimental.pallas import tpu_sc as plsc

_N = 10000
_E = 320000
_H = 128
_R = 8

_L = 16
_NC = 2
_NS = 16
_HC = _H // _L
_RN = _R * _N

_EP_ROWS = 2560
_E_PAD = _EP_ROWS * 128
_KB = 8
_ROWS_T = _EP_ROWS // _NS
_NSB = _ROWS_T // _KB
_ROWS_TC = _EP_ROWS // (_NC * _NS)
_NSB_C = _ROWS_TC // _KB
_ACC_ROWS = _RN + _L
_DUMP = _RN // _NS

_BN = 1000


def _sc_body(do_count, h_tab, src2d, seg2d, zeros_hbm, *rest):
    if do_count:
        (acc_out, cnt_out, acc_sp, ones_v, src_a, src_b, seg_a,
         seg_b, rows_a, rows_b, lsem, gsem, ssem) = rest
    else:
        (acc_out, acc_sp, src_a, src_b, seg_a, seg_b, rows_a,
         rows_b, lsem, gsem, ssem) = rest

    cid = lax.axis_index("c")
    sid = lax.axis_index("s")

    if do_count:
        def init_ones(t, carry):
            ones_v[t] = jnp.ones((_L,), jnp.float32)
            return carry

        lax.fori_loop(0, 128, init_ones, 0)

    def zero_own_slice():
        pltpu.sync_copy(zeros_hbm.at[pl.ds(sid * _DUMP, _DUMP)],
                        acc_sp.at[pl.ds(sid * _DUMP, _DUMP)])

    def drain_scatters():
        for j in range(_KB):
            pltpu.make_async_copy(rows_a.at[pl.ds(j * 128, 128)],
                                  acc_sp.at[seg_a.at[j]], ssem).wait()
        for j in range(_KB):
            pltpu.make_async_copy(rows_b.at[pl.ds(j * 128, 128)],
                                  acc_sp.at[seg_b.at[j]], ssem).wait()

    def drain_cnt_scatters():
        for j in range(2 * _KB):
            pltpu.make_async_copy(ones_v, acc_sp.at[seg_a.at[0]],
                                  ssem).wait()

    zero_own_slice()
    plsc.subcore_barrier()

    if do_count:
        w = cid * _NS + sid

        def cnt_pair(i, carry):
            @pl.when(i > 0)
            def _():
                drain_cnt_scatters()

            ra = w * _ROWS_TC + (2 * i) * _KB
            la = pltpu.async_copy(seg2d.at[pl.ds(ra, _KB)], seg_a, lsem)
            lb = pltpu.async_copy(seg2d.at[pl.ds(ra + _KB, _KB)], seg_b,
                                  lsem)
            la.wait()
            for j in range(_KB):
                pltpu.async_copy(ones_v, acc_sp.at[seg_a.at[j]], ssem,
                                 add=True)
            lb.wait()
            for j in range(_KB):
                pltpu.async_copy(ones_v, acc_sp.at[seg_b.at[j]], ssem,
                                 add=True)
            return carry

        lax.fori_loop(0, _NSB_C // 2, cnt_pair, 0)
        drain_cnt_scatters()
        plsc.subcore_barrier()
        pltpu.sync_copy(
            acc_sp.at[pl.ds(sid * _DUMP, _DUMP)],
            cnt_out.at[pl.ds(cid * _RN + sid * _DUMP, _DUMP)])
        zero_own_slice()
        plsc.subcore_barrier()

    for k in range(_HC // _NC):
        c = cid * (_HC // _NC) + k
        tab_c = h_tab.at[pl.ds(c * _N, _N)]

        def pair_step(i, carry):
            @pl.when(i > 0)
            def _():
                drain_scatters()

            ra = sid * _ROWS_T + (2 * i) * _KB
            rbb = ra + _KB
            la = [pltpu.async_copy(src2d.at[pl.ds(ra, _KB)], src_a, lsem),
                  pltpu.async_copy(seg2d.at[pl.ds(ra, _KB)], seg_a, lsem)]
            lb = [pltpu.async_copy(src2d.at[pl.ds(rbb, _KB)], src_b, lsem),
                  pltpu.async_copy(seg2d.at[pl.ds(rbb, _KB)], seg_b, lsem)]
            for d in la:
                d.wait()
            ga = [
                pltpu.async_copy(tab_c.at[src_a.at[j]],
                                 rows_a.at[pl.ds(j * 128, 128)], gsem)
                for j in range(_KB)
            ]
            for d in lb:
                d.wait()
            for j in range(_KB):
                ga[j].wait()
                pltpu.async_copy(rows_a.at[pl.ds(j * 128, 128)],
                                 acc_sp.at[seg_a.at[j]], ssem, add=True)
            gb = [
                pltpu.async_copy(tab_c.at[src_b.at[j]],
                                 rows_b.at[pl.ds(j * 128, 128)], gsem)
                for j in range(_KB)
            ]
            for j in range(_KB):
                gb[j].wait()
                pltpu.async_copy(rows_b.at[pl.ds(j * 128, 128)],
                                 acc_sp.at[seg_b.at[j]], ssem, add=True)
            return carry

        lax.fori_loop(0, _NSB // 2, pair_step, 0)
        drain_scatters()
        plsc.subcore_barrier()
        pltpu.sync_copy(
            acc_sp.at[pl.ds(sid * _DUMP, _DUMP)],
            acc_out.at[pl.ds(sid * _DUMP, _DUMP), pl.ds(c * _L, _L)])
        if k < _HC // _NC - 1:
            zero_own_slice()
            plsc.subcore_barrier()


def _make_sc_kernel(do_count):
    mesh = plsc.VectorSubcoreMesh(
        core_axis_name="c", subcore_axis_name="s",
        num_cores=_NC, num_subcores=_NS)
    out_type = [jax.ShapeDtypeStruct((_RN, _H), jnp.float32)]
    scratch = [
        pltpu.VMEM_SHARED((_ACC_ROWS, _L), jnp.float32),
    ]
    if do_count:
        out_type.append(jax.ShapeDtypeStruct((_NC * _RN, _L), jnp.float32))
        scratch.append(pltpu.VMEM((128, _L), jnp.float32))
    scratch += [
        pltpu.VMEM((_KB, 128), jnp.int32),
        pltpu.VMEM((_KB, 128), jnp.int32),
        pltpu.VMEM((_KB, 128), jnp.int32),
        pltpu.VMEM((_KB, 128), jnp.int32),
        pltpu.VMEM((_KB * 128, _L), jnp.float32),
        pltpu.VMEM((_KB * 128, _L), jnp.float32),
        pltpu.SemaphoreType.DMA,
        pltpu.SemaphoreType.DMA,
        pltpu.SemaphoreType.DMA,
    ]
    return pl.kernel(
        functools.partial(_sc_body, do_count),
        out_type=tuple(out_type),
        mesh=mesh,
        scratch_types=scratch,
        compiler_params=pltpu.CompilerParams(use_tc_tiling_on_sc=False),
    )


def _dense_body(h_ref, sums_ref, invt_ref, w_ref, root_ref, b_ref, o_ref,
                oc_ref):
    acc = jnp.dot(h_ref[...], root_ref[...],
                  preferred_element_type=jnp.float32) + b_ref[...]
    for r in range(_R):
        mean = sums_ref[r] * invt_ref[:, r:r + 1]
        acc = acc + jnp.dot(mean, w_ref[r],
                            preferred_element_type=jnp.float32)
    res = jnp.maximum(acc, 0.0)
    o_ref[...] = res
    for c in range(_HC):
        oc_ref[c] = res[:, c * _L:(c + 1) * _L]


def _dense_combine(h, sums, invt, w, root, b):
    grid = (_N // _BN,)
    return pl.pallas_call(
        _dense_body,
        grid=grid,
        in_specs=[
            pl.BlockSpec((_BN, _H), lambda i: (i, 0)),
            pl.BlockSpec((_R, _BN, _H), lambda i: (0, i, 0)),
            pl.BlockSpec((_BN, _R), lambda i: (i, 0)),
            pl.BlockSpec((_R, _H, _H), lambda i: (0, 0, 0)),
            pl.BlockSpec((_H, _H), lambda i: (0, 0)),
            pl.BlockSpec((1, _H), lambda i: (0, 0)),
        ],
        out_specs=[
            pl.BlockSpec((_BN, _H), lambda i: (i, 0)),
            pl.BlockSpec((_HC, _BN, _L), lambda i: (0, i, 0)),
        ],
        out_shape=[
            jax.ShapeDtypeStruct((_N, _H), jnp.float32),
            jax.ShapeDtypeStruct((_HC, _N, _L), jnp.float32),
        ],
    )(h, sums, invt, w, root, b.reshape(1, _H))


_sc_agg_count = _make_sc_kernel(True)
_sc_agg = _make_sc_kernel(False)


def kernel(x, edge_index, edge_type, node_emb, w1, root1, b1, w2, root2, b2):
    src = edge_index[0]
    dst = edge_index[1]
    seg = edge_type * _N + dst
    pad = _E_PAD - _E
    src_p = jnp.concatenate([src, jnp.zeros((pad,), jnp.int32)])
    seg_p = jnp.concatenate([seg, jnp.full((pad,), _RN, jnp.int32)])
    src2d = src_p.reshape(_EP_ROWS, 128)
    seg2d = seg_p.reshape(_EP_ROWS, 128)

    h = jnp.take(node_emb, x, axis=0)
    h_tab = h.reshape(_N, _HC, _L).transpose(1, 0, 2).reshape(_HC * _N, _L)
    zeros_sp = jnp.zeros((_RN, _L), jnp.float32)

    invt = None
    for (w, root, b, first) in ((w1, root1, b1, True),
                                (w2, root2, b2, False)):
        if first:
            acc, cnt2 = _sc_agg_count(h_tab, src2d, seg2d, zeros_sp)
            cnt = cnt2[:_RN, 0] + cnt2[_RN:, 0]
            invt = (1.0 / jnp.maximum(cnt, 1.0)).reshape(_R, _N).T
        else:
            (acc,) = _sc_agg(h_tab, src2d, seg2d, zeros_sp)
        sums = acc.reshape(_R, _N, _H)
        h, h_chunks = _dense_combine(h, sums, invt, w, root, b)
        h_tab = h_chunks.reshape(_HC * _N, _L)
    return h
